# baseline (speedup 1.0000x reference)
import functools
import math

import jax
import jax.numpy as jnp
from jax import lax
from jax.experimental import pallas as pl
from jax.experimental.pallas import tpu as pltpu
from jax.experimental.pallas import tpu_sc as plsc

_LANES = 16
_CHUNK_ROWS = 16


def _make_sc_kernel(batch, seq_len, d_model, scale):
    info = plsc.get_sparse_core_info()
    nw = info.num_cores * info.num_subcores
    total_rows = batch * seq_len
    rows_per_w = total_rows // nw
    assert total_rows % nw == 0
    assert rows_per_w % (2 * _CHUNK_ROWS) == 0
    assert seq_len % rows_per_w == 0
    w_per_b = seq_len // rows_per_w
    n_pairs = rows_per_w // _CHUNK_ROWS // 2
    mesh = plsc.VectorSubcoreMesh(core_axis_name="c", subcore_axis_name="s")

    @functools.partial(
        pl.kernel,
        out_type=jax.ShapeDtypeStruct((batch, seq_len, d_model), jnp.float32),
        mesh=mesh,
        scratch_types=[
            pltpu.VMEM((_CHUNK_ROWS, d_model), jnp.float32),
            pltpu.VMEM((_CHUNK_ROWS, d_model), jnp.float32),
            pltpu.VMEM((_CHUNK_ROWS, d_model), jnp.float32),
            pltpu.VMEM((_CHUNK_ROWS, d_model), jnp.float32),
            pltpu.SemaphoreType.DMA,
            pltpu.SemaphoreType.DMA,
            pltpu.SemaphoreType.DMA,
            pltpu.SemaphoreType.DMA,
        ],
        compiler_params=pltpu.CompilerParams(use_tc_tiling_on_sc=True),
    )
    def sc_kernel(x_hbm, pe_hbm, o_hbm, xb0, pb0, xb1, pb1, ld0, ld1, st0, st1):
        wid = lax.axis_index("s") * info.num_cores + lax.axis_index("c")
        b = wid // w_per_b
        s0 = (wid % w_per_b) * rows_per_w

        def start_load(c, xb, pb, sem):
            r = s0 + c * _CHUNK_ROWS
            pltpu.async_copy(x_hbm.at[b, pl.ds(r, _CHUNK_ROWS), :], xb, sem)
            pltpu.async_copy(pe_hbm.at[pl.ds(r, _CHUNK_ROWS), :], pb, sem)

        def wait_load(c, xb, pb, sem):
            r = s0 + c * _CHUNK_ROWS
            pltpu.make_async_copy(x_hbm.at[b, pl.ds(r, _CHUNK_ROWS), :], xb, sem).wait()
            pltpu.make_async_copy(pe_hbm.at[pl.ds(r, _CHUNK_ROWS), :], pb, sem).wait()

        def start_store(c, xb, sem):
            r = s0 + c * _CHUNK_ROWS
            pltpu.async_copy(xb, o_hbm.at[b, pl.ds(r, _CHUNK_ROWS), :], sem)

        def wait_store(c, xb, sem):
            r = s0 + c * _CHUNK_ROWS
            pltpu.make_async_copy(xb, o_hbm.at[b, pl.ds(r, _CHUNK_ROWS), :], sem).wait()

        def compute(xb, pb):
            def row(r, _):
                def body(i, _):
                    sl = pl.ds(i * _LANES, _LANES)
                    xb[r, sl] = xb[r, sl] + pb[r, sl] * scale
                    return ()

                lax.fori_loop(0, d_model // _LANES, body, (), unroll=8)
                return ()

            lax.fori_loop(0, _CHUNK_ROWS, row, ())

        start_load(0, xb0, pb0, ld0)
        start_load(1, xb1, pb1, ld1)

        def step(j, _):
            c0 = 2 * j
            c1 = c0 + 1
            wait_load(c0, xb0, pb0, ld0)
            compute(xb0, pb0)
            start_store(c0, xb0, st0)
            wait_load(c1, xb1, pb1, ld1)
            compute(xb1, pb1)
            start_store(c1, xb1, st1)
            wait_store(c0, xb0, st0)

            @pl.when(j + 1 < n_pairs)
            def _():
                start_load(c0 + 2, xb0, pb0, ld0)

            wait_store(c1, xb1, st1)

            @pl.when(j + 1 < n_pairs)
            def _():
                start_load(c1 + 2, xb1, pb1, ld1)

            return ()

        lax.fori_loop(0, n_pairs, step, ())

    return sc_kernel


def kernel(x, pos_embed):
    batch, seq_len, d_model = x.shape
    scale = math.sqrt(d_model)
    pe = pos_embed[:seq_len]
    sc = _make_sc_kernel(batch, seq_len, d_model, scale)
    return sc(x, pe)


# speedup vs baseline: 5.4088x; 5.4088x over previous
import math

import jax
import jax.numpy as jnp
from jax.experimental import pallas as pl


_BS = 2048


def _pe_add_kernel(x_ref, pe_ref, o_ref, *, scale):
    o_ref[...] = x_ref[...] + pe_ref[...] * scale


def kernel(x, pos_embed):
    batch, seq_len, d_model = x.shape
    scale = math.sqrt(d_model)
    pe = pos_embed[:seq_len]

    bs = min(_BS, seq_len)
    grid = (seq_len // bs, batch)

    return pl.pallas_call(
        lambda xr, pr, orf: _pe_add_kernel(xr, pr, orf, scale=scale),
        grid=grid,
        in_specs=[
            pl.BlockSpec((1, bs, d_model), lambda s, b: (b, s, 0)),
            pl.BlockSpec((bs, d_model), lambda s, b: (s, 0)),
        ],
        out_specs=pl.BlockSpec((1, bs, d_model), lambda s, b: (b, s, 0)),
        out_shape=jax.ShapeDtypeStruct(x.shape, x.dtype),
    )(x, pe)
